# baseline (speedup 1.0000x reference)
import functools

import jax
import jax.numpy as jnp
from jax import lax
from jax.experimental import pallas as pl
from jax.experimental.pallas import tpu as pltpu
from jax.experimental.pallas import tpu_sc as plsc

LC = 1024
CP = 64
NB = 64
HID = 64
AH = 32
TEMP = 4.0
NP = LC * LC // 2
GI = 8


def _node_body(s_ref, wu_ref, bu_ref, wv_ref, bv_ref, wb1_ref, wb2_ref, bb_ref,
               u_ref, v_ref, a_ref, b_ref):
    s = s_ref[...]
    u = jnp.dot(s, wu_ref[...], preferred_element_type=jnp.float32, precision=lax.Precision.HIGHEST) + bu_ref[...]
    v = jnp.dot(s, wv_ref[...], preferred_element_type=jnp.float32, precision=lax.Precision.HIGHEST) + bv_ref[...]
    u_ref[...] = u
    v_ref[...] = v
    a_ref[...] = jnp.dot(u, wb1_ref[...], preferred_element_type=jnp.float32, precision=lax.Precision.HIGHEST) + bb_ref[...]
    b_ref[...] = jnp.dot(v, wb2_ref[...], preferred_element_type=jnp.float32, precision=lax.Precision.HIGHEST)


def _split_bf16(a):
    hi = a.astype(jnp.bfloat16)
    lo = (a - hi.astype(jnp.float32)).astype(jnp.bfloat16)
    return hi, lo


def _dot_x3(a, w_hi, w_lo):
    a_hi, a_lo = _split_bf16(a)
    d = lambda x, y: jnp.dot(x, y, preferred_element_type=jnp.float32)
    return d(a_hi, w_hi) + d(a_hi, w_lo) + d(a_lo, w_hi)


def _dot_x3_t(wt_hi, wt_lo, a):
    a_hi, a_lo = _split_bf16(a)
    dims = (((1,), (1,)), ((), ()))
    d = lambda x, y: lax.dot_general(x, y, dims,
                                     preferred_element_type=jnp.float32)
    return d(wt_hi, a_hi) + d(wt_lo, a_hi) + d(wt_hi, a_lo)


def _dense_body(z_ref, d_ref, u_ref, a_ref, v_ref, b_ref,
                w1h_ref, w1l_ref, bd_ref, g_ref, be_ref, ba1_ref, wa2_ref,
                w2h_ref, w2l_ref, s_out_ref, v_out_ref):
    rows = GI * LC
    zb = z_ref[...].reshape(rows, CP)
    db = d_ref[...].reshape(rows, NB)
    ub = u_ref[...]
    ab = a_ref[...]
    vf = v_ref[...]
    bf = b_ref[...]
    had = (ub[:, None, :] * vf[None, :, :]).reshape(rows, HID)
    x1 = jnp.concatenate([had, db], axis=1)
    s1 = _dot_x3(x1, w1h_ref[...], w1l_ref[...])
    zh = zb + s1[:, :CP] + (ab[:, None, :] + bf[None, :, :]).reshape(rows, CP)
    dp = s1[:, CP:] + bd_ref[...]
    hp = jnp.concatenate([zh, dp], axis=1)
    mu = jnp.sum(hp, axis=1, keepdims=True) * (1.0 / (2 * CP))
    hc = hp - mu
    var = jnp.sum(hc * hc, axis=1, keepdims=True) * (1.0 / (2 * CP))
    h = hc * lax.rsqrt(var + 1e-5) * g_ref[...] + be_ref[...]
    s2t = _dot_x3_t(w2h_ref[...], w2l_ref[...], h)
    tt = jnp.maximum(s2t[:AH, :] + ba1_ref[...], 0.0)
    sct = jnp.sum(tt * wa2_ref[...], axis=0, keepdims=True)
    vlt = s2t[AH:AH + 1, :]
    s_out_ref[...] = sct.reshape(rows)
    v_out_ref[...] = vlt.reshape(rows)


def _block_diag(a, b):
    za = jnp.zeros_like(a)
    return jnp.concatenate(
        [jnp.concatenate([a, za], axis=1), jnp.concatenate([za, b], axis=1)],
        axis=0)


_NCH = 64
_CH = NP // _NCH


def _pool_sum_body(s_ref, v_ref, sum_ref, psum_ref, acc):
    i = pl.program_id(0)

    @pl.when(i == 0)
    def _():
        acc[0] = 0.0
        acc[1] = 0.0

    e = jnp.exp(s_ref[...] * (1.0 / TEMP))
    acc[0] += jnp.sum(e)
    acc[1] += jnp.sum(e * v_ref[...])

    @pl.when(i == _NCH - 1)
    def _():
        sum_ref[0, 0] = acc[0]
        psum_ref[0, 0] = acc[1]


def _pool_w_body(s_ref, sum_ref, psum_ref, bval_ref, w_ref, p_ref):
    inv = 1.0 / sum_ref[0, 0]
    w_ref[...] = jnp.exp(s_ref[...] * (1.0 / TEMP)) * inv

    @pl.when(pl.program_id(0) == 0)
    def _():
        p_ref[0, 0] = psum_ref[0, 0] * inv + bval_ref[0, 0]


def _sc_gather(scores_flat, vals_flat, idx):
    info = plsc.get_sparse_core_info()
    nc, ns = info.num_cores, info.num_subcores
    nw = nc * ns
    bpw = NP // nw
    mesh = plsc.VectorSubcoreMesh(core_axis_name="c", subcore_axis_name="s")

    @functools.partial(
        pl.kernel, mesh=mesh,
        out_type=[jax.ShapeDtypeStruct((NP,), jnp.float32),
                  jax.ShapeDtypeStruct((NP,), jnp.float32)],
        scratch_types=[pltpu.VMEM((bpw,), jnp.int32),
                       pltpu.VMEM((bpw,), jnp.float32),
                       pltpu.VMEM((bpw,), jnp.float32),
                       pltpu.SemaphoreType.DMA,
                       pltpu.SemaphoreType.DMA],
    )
    def gather_k(s_hbm, v_hbm, idx_hbm, os_hbm, ov_hbm, idx_v, sv, vv, sem1, sem2):
        wid = lax.axis_index("s") * nc + lax.axis_index("c")
        base = wid * bpw
        pltpu.sync_copy(idx_hbm.at[pl.ds(base, bpw)], idx_v)
        c1 = pltpu.async_copy(s_hbm.at[idx_v], sv, sem1)
        c2 = pltpu.async_copy(v_hbm.at[idx_v], vv, sem2)
        c1.wait()
        c2.wait()
        pltpu.sync_copy(sv, os_hbm.at[pl.ds(base, bpw)])
        pltpu.sync_copy(vv, ov_hbm.at[pl.ds(base, bpw)])

    return gather_k(scores_flat, vals_flat, idx)


def kernel(z, s_proxy, dist_bins, pd_flat_idx, pd_pairs,
           Wu, bu, Wv, bv, Wb, bb, Wd, bd, gamma, beta,
           Wa1, ba1, Wa2, ba2, Wval, bval):
    f32 = jnp.float32
    row = lambda x: x.reshape(1, -1)

    u, v, a_pre, b_pre = pl.pallas_call(
        _node_body,
        out_shape=[jax.ShapeDtypeStruct((LC, HID), f32)] * 2
        + [jax.ShapeDtypeStruct((LC, CP), f32)] * 2,
    )(s_proxy, Wu, row(bu), Wv, row(bv), Wb[:HID], Wb[HID:2 * HID], row(bb))

    nsteps = LC // GI
    full = lambda shp: pl.BlockSpec(shp, lambda i: (0,) * len(shp))
    scores, vals = pl.pallas_call(
        _dense_body,
        grid=(nsteps,),
        in_specs=[
            pl.BlockSpec((GI, LC, CP), lambda i: (i, 0, 0)),
            pl.BlockSpec((GI, LC, NB), lambda i: (i, 0, 0)),
            pl.BlockSpec((GI, HID), lambda i: (i, 0)),
            pl.BlockSpec((GI, CP), lambda i: (i, 0)),
            full((LC, HID)),
            full((LC, CP)),
            full((2 * CP, 2 * CP)),
            full((2 * CP, 2 * CP)),
            full((1, CP)),
            full((1, 2 * CP)),
            full((1, 2 * CP)),
            full((AH, 1)),
            full((AH, 1)),
            full((AH + 1, 2 * CP)),
            full((AH + 1, 2 * CP)),
        ],
        out_specs=[pl.BlockSpec((GI * LC,), lambda i: (i,)),
                   pl.BlockSpec((GI * LC,), lambda i: (i,))],
        out_shape=[jax.ShapeDtypeStruct((LC * LC,), f32)] * 2,
        compiler_params=pltpu.CompilerParams(
            dimension_semantics=("arbitrary",)),
    )(z, dist_bins, u, a_pre, v, b_pre,
      *_split_bf16(_block_diag(Wb[2 * HID:], Wd)), row(bd), row(gamma),
      row(beta), ba1.reshape(AH, 1), Wa2.reshape(AH, 1),
      *_split_bf16(jnp.concatenate([Wa1, Wval[:, None]], axis=1).T))

    sc_pd, val_pd = _sc_gather(scores, vals, pd_flat_idx.astype(jnp.int32))

    esum, epsum = pl.pallas_call(
        _pool_sum_body,
        grid=(_NCH,),
        in_specs=[pl.BlockSpec((_CH,), lambda i: (i,)),
                  pl.BlockSpec((_CH,), lambda i: (i,))],
        out_specs=[pl.BlockSpec(memory_space=pltpu.SMEM),
                   pl.BlockSpec(memory_space=pltpu.SMEM)],
        out_shape=[jax.ShapeDtypeStruct((1, 1), f32)] * 2,
        scratch_shapes=[pltpu.SMEM((2,), f32)],
    )(sc_pd, val_pd)

    w2, pooled = pl.pallas_call(
        _pool_w_body,
        grid=(_NCH,),
        in_specs=[pl.BlockSpec((_CH,), lambda i: (i,)),
                  pl.BlockSpec(memory_space=pltpu.SMEM),
                  pl.BlockSpec(memory_space=pltpu.SMEM),
                  pl.BlockSpec(memory_space=pltpu.SMEM)],
        out_specs=[pl.BlockSpec((_CH,), lambda i: (i,)),
                   pl.BlockSpec(memory_space=pltpu.SMEM)],
        out_shape=[jax.ShapeDtypeStruct((NP,), f32),
                   jax.ShapeDtypeStruct((1, 1), f32)],
    )(sc_pd, esum, epsum, bval.reshape(1, 1))

    return pooled[0, 0], w2


# speedup vs baseline: 2.3247x; 2.3247x over previous
import functools

import jax
import jax.numpy as jnp
from jax import lax
from jax.experimental import pallas as pl
from jax.experimental.pallas import tpu as pltpu
from jax.experimental.pallas import tpu_sc as plsc

LC = 1024
CP = 64
NB = 64
HID = 64
AH = 32
TEMP = 4.0
NP = LC * LC // 2
GI = 8


def _split_bf16(a):
    hi = a.astype(jnp.bfloat16)
    lo = (a - hi.astype(jnp.float32)).astype(jnp.bfloat16)
    return hi, lo


def _dot_x3(w_hi, w_lo, a):
    a_hi, a_lo = _split_bf16(a)
    d = lambda x, y: jnp.dot(x, y, preferred_element_type=jnp.float32)
    return d(w_hi, a_hi) + d(w_lo, a_hi) + d(w_hi, a_lo)


def _node_body(s_ref, wu_ref, bu_ref, wv_ref, bv_ref, wb1_ref, wb2_ref, bb_ref,
               u_ref, v_ref, a_ref, b_ref):
    s = s_ref[...]
    hp = lax.Precision.HIGHEST
    u = jnp.dot(s, wu_ref[...], preferred_element_type=jnp.float32, precision=hp) + bu_ref[...]
    v = jnp.dot(s, wv_ref[...], preferred_element_type=jnp.float32, precision=hp) + bv_ref[...]
    a = jnp.dot(u, wb1_ref[...], preferred_element_type=jnp.float32, precision=hp) + bb_ref[...]
    b = jnp.dot(v, wb2_ref[...], preferred_element_type=jnp.float32, precision=hp)
    u_ref[...] = u
    v_ref[...] = v.T
    a_ref[...] = a
    b_ref[...] = b.T


def _dense_body(z_ref, d_ref, u_ref, a_ref, v_ref, b_ref,
                w1h_ref, w1l_ref, bd_ref, g_ref, be_ref, ba1_ref, wa2_ref,
                w2h_ref, w2l_ref, s_out_ref, v_out_ref):
    vt = v_ref[...]
    bt = b_ref[...]
    ucols = jnp.transpose(u_ref[...])
    acols = jnp.transpose(a_ref[...])
    scs = []
    vls = []
    for g in range(GI):
        zb = z_ref[g]
        db = d_ref[g]
        ucol = ucols[:, g:g + 1]
        acol = acols[:, g:g + 1]
        had = ucol * vt
        x1 = jnp.concatenate([had, db], axis=0)
        s1 = _dot_x3(w1h_ref[...], w1l_ref[...], x1)
        zh = zb + s1[:CP, :] + acol + bt
        dp = s1[CP:, :] + bd_ref[...]
        hp = jnp.concatenate([zh, dp], axis=0)
        mu = jnp.sum(hp, axis=0, keepdims=True) * (1.0 / (2 * CP))
        hc = hp - mu
        var = jnp.sum(hc * hc, axis=0, keepdims=True) * (1.0 / (2 * CP))
        h = hc * lax.rsqrt(var + 1e-5) * g_ref[...] + be_ref[...]
        s2 = _dot_x3(w2h_ref[...], w2l_ref[...], h)
        t = jnp.maximum(s2[:AH, :] + ba1_ref[...], 0.0)
        scs.append(jnp.sum(t * wa2_ref[...], axis=0, keepdims=True))
        vls.append(s2[AH:AH + 1, :])
    s_out_ref[...] = jnp.concatenate(scs, axis=0).reshape(GI * LC)
    v_out_ref[...] = jnp.concatenate(vls, axis=0).reshape(GI * LC)


def _block_diag(a, b):
    za = jnp.zeros_like(a)
    return jnp.concatenate(
        [jnp.concatenate([a, za], axis=1), jnp.concatenate([za, b], axis=1)],
        axis=0)


def _pool_body(s_ref, v_ref, bval_ref, w_ref, p_ref):
    s = s_ref[...]
    v = v_ref[...]
    m = jnp.max(s)
    e = jnp.exp((s - m) * (1.0 / TEMP))
    tot = jnp.sum(e)
    w = e / tot
    w_ref[...] = w
    p_ref[0, 0] = jnp.sum(w * v) + bval_ref[0, 0]


def _sc_gather(scores_flat, vals_flat, idx):
    info = plsc.get_sparse_core_info()
    nc, ns = info.num_cores, info.num_subcores
    nw = nc * ns
    bpw = NP // nw
    mesh = plsc.VectorSubcoreMesh(core_axis_name="c", subcore_axis_name="s")

    @functools.partial(
        pl.kernel, mesh=mesh,
        out_type=[jax.ShapeDtypeStruct((NP,), jnp.float32),
                  jax.ShapeDtypeStruct((NP,), jnp.float32)],
        scratch_types=[pltpu.VMEM((bpw,), jnp.int32),
                       pltpu.VMEM((bpw,), jnp.float32),
                       pltpu.VMEM((bpw,), jnp.float32),
                       pltpu.SemaphoreType.DMA,
                       pltpu.SemaphoreType.DMA],
    )
    def gather_k(s_hbm, v_hbm, idx_hbm, os_hbm, ov_hbm, idx_v, sv, vv, sem1, sem2):
        wid = lax.axis_index("s") * nc + lax.axis_index("c")
        base = wid * bpw
        pltpu.sync_copy(idx_hbm.at[pl.ds(base, bpw)], idx_v)
        c1 = pltpu.async_copy(s_hbm.at[idx_v], sv, sem1)
        c2 = pltpu.async_copy(v_hbm.at[idx_v], vv, sem2)
        c1.wait()
        c2.wait()
        pltpu.sync_copy(sv, os_hbm.at[pl.ds(base, bpw)])
        pltpu.sync_copy(vv, ov_hbm.at[pl.ds(base, bpw)])

    return gather_k(scores_flat, vals_flat, idx)


def kernel(z, s_proxy, dist_bins, pd_flat_idx, pd_pairs,
           Wu, bu, Wv, bv, Wb, bb, Wd, bd, gamma, beta,
           Wa1, ba1, Wa2, ba2, Wval, bval):
    f32 = jnp.float32
    row = lambda x: x.reshape(1, -1)
    col = lambda x: x.reshape(-1, 1)

    ut, vt, at, bt = pl.pallas_call(
        _node_body,
        out_shape=[jax.ShapeDtypeStruct((LC, HID), f32),
                   jax.ShapeDtypeStruct((HID, LC), f32),
                   jax.ShapeDtypeStruct((LC, CP), f32),
                   jax.ShapeDtypeStruct((CP, LC), f32)],
    )(s_proxy, Wu, row(bu), Wv, row(bv), Wb[:HID], Wb[HID:2 * HID], row(bb))

    nsteps = LC // GI
    full = lambda shp: pl.BlockSpec(shp, lambda i: (0,) * len(shp))
    scores, vals = pl.pallas_call(
        _dense_body,
        grid=(nsteps,),
        in_specs=[
            pl.BlockSpec((GI, CP, LC), lambda i: (i, 0, 0)),
            pl.BlockSpec((GI, NB, LC), lambda i: (i, 0, 0)),
            pl.BlockSpec((GI, HID), lambda i: (i, 0)),
            pl.BlockSpec((GI, CP), lambda i: (i, 0)),
            full((HID, LC)),
            full((CP, LC)),
            full((2 * CP, 2 * CP)),
            full((2 * CP, 2 * CP)),
            full((CP, 1)),
            full((2 * CP, 1)),
            full((2 * CP, 1)),
            full((AH, 1)),
            full((AH, 1)),
            full((AH + 1, 2 * CP)),
            full((AH + 1, 2 * CP)),
        ],
        out_specs=[pl.BlockSpec((GI * LC,), lambda i: (i,)),
                   pl.BlockSpec((GI * LC,), lambda i: (i,))],
        out_shape=[jax.ShapeDtypeStruct((LC * LC,), f32)] * 2,
        compiler_params=pltpu.CompilerParams(
            dimension_semantics=("arbitrary",)),
    )(z.transpose(0, 2, 1), dist_bins.transpose(0, 2, 1), ut, at, vt, bt,
      *_split_bf16(_block_diag(Wb[2 * HID:], Wd).T), col(bd), col(gamma),
      col(beta), col(ba1), col(Wa2),
      *_split_bf16(jnp.concatenate([Wa1, Wval[:, None]], axis=1).T))

    sc_pd, val_pd = _sc_gather(scores, vals, pd_flat_idx.astype(jnp.int32))

    w2, pooled = pl.pallas_call(
        _pool_body,
        in_specs=[pl.BlockSpec((NP // LC, LC), lambda: (0, 0)),
                  pl.BlockSpec((NP // LC, LC), lambda: (0, 0)),
                  pl.BlockSpec(memory_space=pltpu.SMEM)],
        out_specs=[pl.BlockSpec((NP // LC, LC), lambda: (0, 0)),
                   pl.BlockSpec(memory_space=pltpu.SMEM)],
        out_shape=[jax.ShapeDtypeStruct((NP // LC, LC), f32),
                   jax.ShapeDtypeStruct((1, 1), f32)],
    )(sc_pd.reshape(NP // LC, LC), val_pd.reshape(NP // LC, LC),
      bval.reshape(1, 1))

    return pooled[0, 0], w2.reshape(-1)


# speedup vs baseline: 2.3816x; 1.0245x over previous
import functools

import jax
import jax.numpy as jnp
from jax import lax
from jax.experimental import pallas as pl
from jax.experimental.pallas import tpu as pltpu
from jax.experimental.pallas import tpu_sc as plsc

LC = 1024
CP = 64
NB = 64
HID = 64
AH = 32
TEMP = 4.0
NP = LC * LC // 2
GI = 16


def _split_bf16(a):
    hi = a.astype(jnp.bfloat16)
    lo = (a - hi.astype(jnp.float32)).astype(jnp.bfloat16)
    return hi, lo


def _dot_x3(w_hi, w_lo, a):
    a_hi, a_lo = _split_bf16(a)
    d = lambda x, y: jnp.dot(x, y, preferred_element_type=jnp.float32)
    return d(w_hi, a_hi) + d(w_lo, a_hi) + d(w_hi, a_lo)


def _node_body(s_ref, wu_ref, bu_ref, wv_ref, bv_ref, wb1_ref, wb2_ref, bb_ref,
               u_ref, v_ref, a_ref, b_ref):
    s = s_ref[...]
    hp = lax.Precision.HIGHEST
    u = jnp.dot(s, wu_ref[...], preferred_element_type=jnp.float32, precision=hp) + bu_ref[...]
    v = jnp.dot(s, wv_ref[...], preferred_element_type=jnp.float32, precision=hp) + bv_ref[...]
    a = jnp.dot(u, wb1_ref[...], preferred_element_type=jnp.float32, precision=hp) + bb_ref[...]
    b = jnp.dot(v, wb2_ref[...], preferred_element_type=jnp.float32, precision=hp)
    u_ref[...] = u
    v_ref[...] = v.T
    a_ref[...] = a
    b_ref[...] = b.T


def _dense_body(z_ref, d_ref, u_ref, a_ref, v_ref, b_ref,
                w1h_ref, w1l_ref, bd_ref, g_ref, be_ref, ba1_ref, wa2_ref,
                w2h_ref, w2l_ref, s_out_ref, v_out_ref):
    vt = v_ref[...]
    bt = b_ref[...]
    ucols = jnp.transpose(u_ref[...])
    acols = jnp.transpose(a_ref[...])
    scs = []
    vls = []
    for g in range(GI):
        zb = z_ref[g]
        db = d_ref[g]
        ucol = ucols[:, g:g + 1]
        acol = acols[:, g:g + 1]
        had = ucol * vt
        x1 = jnp.concatenate([had, db], axis=0)
        s1 = _dot_x3(w1h_ref[...], w1l_ref[...], x1)
        zh = zb + s1[:CP, :] + acol + bt
        dp = s1[CP:, :] + bd_ref[...]
        hp = jnp.concatenate([zh, dp], axis=0)
        mu = jnp.sum(hp, axis=0, keepdims=True) * (1.0 / (2 * CP))
        hc = hp - mu
        var = jnp.sum(hc * hc, axis=0, keepdims=True) * (1.0 / (2 * CP))
        h = hc * lax.rsqrt(var + 1e-5) * g_ref[...] + be_ref[...]
        s2 = _dot_x3(w2h_ref[...], w2l_ref[...], h)
        t = jnp.maximum(s2[:AH, :] + ba1_ref[...], 0.0)
        scs.append(jnp.sum(t * wa2_ref[...], axis=0, keepdims=True))
        vls.append(s2[AH:AH + 1, :])
    s_out_ref[...] = jnp.concatenate(scs, axis=0).reshape(GI * LC)
    v_out_ref[...] = jnp.concatenate(vls, axis=0).reshape(GI * LC)


def _block_diag(a, b):
    za = jnp.zeros_like(a)
    return jnp.concatenate(
        [jnp.concatenate([a, za], axis=1), jnp.concatenate([za, b], axis=1)],
        axis=0)


def _pool_body(s_ref, v_ref, bval_ref, w_ref, p_ref):
    s = s_ref[...]
    v = v_ref[...]
    m = jnp.max(s)
    e = jnp.exp((s - m) * (1.0 / TEMP))
    tot = jnp.sum(e)
    w = e / tot
    w_ref[...] = w
    p_ref[0, 0] = jnp.sum(w * v) + bval_ref[0, 0]


def _sc_gather(scores_flat, vals_flat, idx):
    info = plsc.get_sparse_core_info()
    nc, ns = info.num_cores, info.num_subcores
    nw = nc * ns
    bpw = NP // nw
    mesh = plsc.VectorSubcoreMesh(core_axis_name="c", subcore_axis_name="s")

    @functools.partial(
        pl.kernel, mesh=mesh,
        out_type=[jax.ShapeDtypeStruct((NP,), jnp.float32),
                  jax.ShapeDtypeStruct((NP,), jnp.float32)],
        scratch_types=[pltpu.VMEM((bpw,), jnp.int32),
                       pltpu.VMEM((bpw,), jnp.float32),
                       pltpu.VMEM((bpw,), jnp.float32),
                       pltpu.SemaphoreType.DMA,
                       pltpu.SemaphoreType.DMA],
    )
    def gather_k(s_hbm, v_hbm, idx_hbm, os_hbm, ov_hbm, idx_v, sv, vv, sem1, sem2):
        wid = lax.axis_index("s") * nc + lax.axis_index("c")
        base = wid * bpw
        pltpu.sync_copy(idx_hbm.at[pl.ds(base, bpw)], idx_v)
        c1 = pltpu.async_copy(s_hbm.at[idx_v], sv, sem1)
        c2 = pltpu.async_copy(v_hbm.at[idx_v], vv, sem2)
        c1.wait()
        c2.wait()
        pltpu.sync_copy(sv, os_hbm.at[pl.ds(base, bpw)])
        pltpu.sync_copy(vv, ov_hbm.at[pl.ds(base, bpw)])

    return gather_k(scores_flat, vals_flat, idx)


def kernel(z, s_proxy, dist_bins, pd_flat_idx, pd_pairs,
           Wu, bu, Wv, bv, Wb, bb, Wd, bd, gamma, beta,
           Wa1, ba1, Wa2, ba2, Wval, bval):
    f32 = jnp.float32
    row = lambda x: x.reshape(1, -1)
    col = lambda x: x.reshape(-1, 1)

    ut, vt, at, bt = pl.pallas_call(
        _node_body,
        out_shape=[jax.ShapeDtypeStruct((LC, HID), f32),
                   jax.ShapeDtypeStruct((HID, LC), f32),
                   jax.ShapeDtypeStruct((LC, CP), f32),
                   jax.ShapeDtypeStruct((CP, LC), f32)],
    )(s_proxy, Wu, row(bu), Wv, row(bv), Wb[:HID], Wb[HID:2 * HID], row(bb))

    nsteps = LC // GI
    full = lambda shp: pl.BlockSpec(shp, lambda i: (0,) * len(shp))
    scores, vals = pl.pallas_call(
        _dense_body,
        grid=(nsteps,),
        in_specs=[
            pl.BlockSpec((GI, CP, LC), lambda i: (i, 0, 0)),
            pl.BlockSpec((GI, NB, LC), lambda i: (i, 0, 0)),
            pl.BlockSpec((GI, HID), lambda i: (i, 0)),
            pl.BlockSpec((GI, CP), lambda i: (i, 0)),
            full((HID, LC)),
            full((CP, LC)),
            full((2 * CP, 2 * CP)),
            full((2 * CP, 2 * CP)),
            full((CP, 1)),
            full((2 * CP, 1)),
            full((2 * CP, 1)),
            full((AH, 1)),
            full((AH, 1)),
            full((AH + 1, 2 * CP)),
            full((AH + 1, 2 * CP)),
        ],
        out_specs=[pl.BlockSpec((GI * LC,), lambda i: (i,)),
                   pl.BlockSpec((GI * LC,), lambda i: (i,))],
        out_shape=[jax.ShapeDtypeStruct((LC * LC,), f32)] * 2,
        compiler_params=pltpu.CompilerParams(
            dimension_semantics=("arbitrary",)),
    )(z.transpose(0, 2, 1), dist_bins.transpose(0, 2, 1), ut, at, vt, bt,
      *_split_bf16(_block_diag(Wb[2 * HID:], Wd).T), col(bd), col(gamma),
      col(beta), col(ba1), col(Wa2),
      *_split_bf16(jnp.concatenate([Wa1, Wval[:, None]], axis=1).T))

    sc_pd, val_pd = _sc_gather(scores, vals, pd_flat_idx.astype(jnp.int32))

    w2, pooled = pl.pallas_call(
        _pool_body,
        in_specs=[pl.BlockSpec((NP // LC, LC), lambda: (0, 0)),
                  pl.BlockSpec((NP // LC, LC), lambda: (0, 0)),
                  pl.BlockSpec(memory_space=pltpu.SMEM)],
        out_specs=[pl.BlockSpec((NP // LC, LC), lambda: (0, 0)),
                   pl.BlockSpec(memory_space=pltpu.SMEM)],
        out_shape=[jax.ShapeDtypeStruct((NP // LC, LC), f32),
                   jax.ShapeDtypeStruct((1, 1), f32)],
    )(sc_pd.reshape(NP // LC, LC), val_pd.reshape(NP // LC, LC),
      bval.reshape(1, 1))

    return pooled[0, 0], w2.reshape(-1)
